# baseline (speedup 1.0000x reference)
import functools

import jax
import jax.numpy as jnp
from jax.experimental import pallas as pl
from jax.experimental.pallas import tpu as pltpu

D_MODEL = 128
N_MEM = 100000
N_B = 128
TOPK = 10
BLKM = 2048
NBLK = (N_MEM + BLKM - 1) // BLKM

_NEG = float("-inf")
_BIGI = 2**30
_HI = jax.lax.Precision.HIGHEST


def _scan_kernel(gf_ref, gn_ref, pat_ref, pn_ref,
                 mW1_ref, mb1_ref, mW2_ref, mb2_ref, mW3_ref, mb3_ref,
                 sW1_ref, sb1_ref, sW2_ref, sb2_ref, sW3_ref, sb3_ref,
                 vals_out, idxs_out, match_out, sparams_out,
                 vals_scr, idxs_scr):
    j = pl.program_id(0)

    @pl.when(j == 0)
    def _init():
        gf = gf_ref[...]
        vals_scr[...] = jnp.full((N_B, 128), _NEG, jnp.float32)
        idxs_scr[...] = jnp.full((N_B, 128), _BIGI, jnp.int32)
        gelu = lambda x: 0.5 * x * (1.0 + jax.lax.erf(x * 0.7071067811865476))
        mf = gelu(jnp.dot(gf, mW1_ref[...], preferred_element_type=jnp.float32, precision=_HI) + mb1_ref[...])
        mf = gelu(jnp.dot(mf, mW2_ref[...], preferred_element_type=jnp.float32, precision=_HI) + mb2_ref[...])
        match = jnp.dot(mf, mW3_ref[...], preferred_element_type=jnp.float32, precision=_HI) + mb3_ref[...]
        match_out[...] = match
        syn = jnp.concatenate([gf, match], axis=1)
        sp = gelu(jnp.dot(syn, sW1_ref[...], preferred_element_type=jnp.float32, precision=_HI) + sb1_ref[...])
        sp = gelu(jnp.dot(sp, sW2_ref[...], preferred_element_type=jnp.float32, precision=_HI) + sb2_ref[...])
        sparams_out[...] = jnp.dot(sp, sW3_ref[...], preferred_element_type=jnp.float32, precision=_HI) + sb3_ref[...]

    pat = pat_ref[...]
    gf = gf_ref[...]
    dots = jax.lax.dot_general(gf, pat, (((1,), (1,)), ((), ())),
                               preferred_element_type=jnp.float32)
    denom = jnp.maximum(gn_ref[...] * pn_ref[...], 1e-8)
    sims = dots / denom
    col = j * BLKM + jax.lax.broadcasted_iota(jnp.int32, (N_B, BLKM), 1)
    valid = col < N_MEM
    sims = jnp.where(valid, sims, _NEG)
    col = jnp.where(valid, col, _BIGI)

    comb_v = jnp.concatenate([vals_scr[...], sims], axis=1)
    comb_i = jnp.concatenate([idxs_scr[...], col], axis=1)
    new_v = jnp.full((N_B, 128), _NEG, jnp.float32)
    new_i = jnp.full((N_B, 128), _BIGI, jnp.int32)
    out_lane = jax.lax.broadcasted_iota(jnp.int32, (N_B, 128), 1)
    for k in range(TOPK):
        m = jnp.max(comb_v, axis=1, keepdims=True)
        is_m = comb_v == m
        sel = jnp.min(jnp.where(is_m, comb_i, _BIGI), axis=1, keepdims=True)
        comb_v = jnp.where(comb_i == sel, _NEG, comb_v)
        new_v = jnp.where(out_lane == k, m, new_v)
        new_i = jnp.where(out_lane == k, sel, new_i)
    vals_scr[...] = new_v
    idxs_scr[...] = new_i

    @pl.when(j == NBLK - 1)
    def _fin():
        vals_out[...] = vals_scr[...]
        idxs_out[...] = idxs_scr[...]


def _gather_kernel(idx_ref, cat_ref, comp_ref, cat_out, comp_out):
    j = pl.program_id(0)

    @pl.when(j == 0)
    def _init():
        cat_out[...] = jnp.zeros_like(cat_out)
        comp_out[...] = jnp.zeros_like(comp_out)

    base = j * BLKM
    col = base + jax.lax.broadcasted_iota(jnp.int32, (N_B, BLKM), 1)
    idx = idx_ref[...]
    onehot = jnp.zeros((N_B, BLKM), jnp.float32)
    for k in range(TOPK):
        onehot = onehot + (idx[:, k:k + 1] == col).astype(jnp.float32)
    row = jax.lax.broadcasted_iota(jnp.int32, (BLKM, 1), 0) + base
    rvalid = row < N_MEM
    cat = jnp.where(rvalid, cat_ref[...], 0.0)
    comp = jnp.where(rvalid, comp_ref[...], 0.0)
    cat_out[...] += jnp.dot(onehot, cat, preferred_element_type=jnp.float32, precision=_HI)
    comp_out[...] += jnp.dot(onehot, comp, preferred_element_type=jnp.float32, precision=_HI)

    @pl.when(j == NBLK - 1)
    def _fin():
        cat_out[...] = cat_out[...] * (1.0 / TOPK)
        comp_out[...] = comp_out[...] * (1.0 / TOPK)


@functools.partial(jax.jit, static_argnames=())
def kernel(features, strategic_patterns, pattern_categories, pattern_complexity,
           mW1, mb1, mW2, mb2, mW3, mb3, sW1, sb1, sW2, sb2, sW3, sb3):
    global_features = jnp.mean(features, axis=(2, 3))
    gn = jnp.linalg.norm(global_features, axis=1, keepdims=True)
    pn = jnp.linalg.norm(strategic_patterns, axis=1, keepdims=True)
    pnT = pn.T

    bias2 = lambda b: b.reshape(1, -1)
    full = lambda shape: pl.BlockSpec(shape, lambda j: tuple(0 for _ in shape))

    vals, idxs, match, sparams = pl.pallas_call(
        _scan_kernel,
        grid=(NBLK,),
        in_specs=[
            full((N_B, D_MODEL)),
            full((N_B, 1)),
            pl.BlockSpec((BLKM, D_MODEL), lambda j: (j, 0)),
            pl.BlockSpec((1, BLKM), lambda j: (0, j)),
            full((D_MODEL, 256)), full((1, 256)),
            full((256, D_MODEL)), full((1, D_MODEL)),
            full((D_MODEL, 128)), full((1, 128)),
            full((256, 256)), full((1, 256)),
            full((256, D_MODEL)), full((1, D_MODEL)),
            full((D_MODEL, 64)), full((1, 64)),
        ],
        out_specs=[
            full((N_B, 128)), full((N_B, 128)),
            full((N_B, 128)), full((N_B, 64)),
        ],
        out_shape=[
            jax.ShapeDtypeStruct((N_B, 128), jnp.float32),
            jax.ShapeDtypeStruct((N_B, 128), jnp.int32),
            jax.ShapeDtypeStruct((N_B, 128), jnp.float32),
            jax.ShapeDtypeStruct((N_B, 64), jnp.float32),
        ],
        scratch_shapes=[
            pltpu.VMEM((N_B, 128), jnp.float32),
            pltpu.VMEM((N_B, 128), jnp.int32),
        ],
    )(global_features, gn, strategic_patterns, pnT,
      mW1, bias2(mb1), mW2, bias2(mb2), mW3, bias2(mb3),
      sW1, bias2(sb1), sW2, bias2(sb2), sW3, bias2(sb3))

    cat_w, comp_w = pl.pallas_call(
        _gather_kernel,
        grid=(NBLK,),
        in_specs=[
            full((N_B, 128)),
            pl.BlockSpec((BLKM, 32), lambda j: (j, 0)),
            pl.BlockSpec((BLKM, 16), lambda j: (j, 0)),
        ],
        out_specs=[full((N_B, 32)), full((N_B, 16))],
        out_shape=[
            jax.ShapeDtypeStruct((N_B, 32), jnp.float32),
            jax.ShapeDtypeStruct((N_B, 16), jnp.float32),
        ],
    )(idxs, pattern_categories, pattern_complexity)

    return (vals[:, :TOPK], idxs[:, :TOPK], cat_w, comp_w, sparams, match)


# speedup vs baseline: 1.3738x; 1.3738x over previous
import functools

import jax
import jax.numpy as jnp
from jax import lax
from jax.experimental import pallas as pl
from jax.experimental.pallas import tpu as pltpu
from jax.experimental.pallas import tpu_sc as plsc

D_MODEL = 128
N_MEM = 100000
N_B = 128
TOPK = 10
BLKM = 2048
NBLK = (N_MEM + BLKM - 1) // BLKM

_NEG = float("-inf")
_BIGI = 2**30
_HI = jax.lax.Precision.HIGHEST


def _scan_kernel(gf_ref, gn_ref, pat_ref, pn_ref,
                 mW1_ref, mb1_ref, mW2_ref, mb2_ref, mW3_ref, mb3_ref,
                 sW1_ref, sb1_ref, sW2_ref, sb2_ref, sW3_ref, sb3_ref,
                 vals_out, idxs_out, match_out, sparams_out,
                 vals_scr, idxs_scr):
    j = pl.program_id(0)

    @pl.when(j == 0)
    def _init():
        gf = gf_ref[...]
        vals_scr[...] = jnp.full((N_B, 128), _NEG, jnp.float32)
        idxs_scr[...] = jnp.full((N_B, 128), _BIGI, jnp.int32)
        gelu = lambda x: 0.5 * x * (1.0 + jax.lax.erf(x * 0.7071067811865476))
        mf = gelu(jnp.dot(gf, mW1_ref[...], preferred_element_type=jnp.float32, precision=_HI) + mb1_ref[...])
        mf = gelu(jnp.dot(mf, mW2_ref[...], preferred_element_type=jnp.float32, precision=_HI) + mb2_ref[...])
        match = jnp.dot(mf, mW3_ref[...], preferred_element_type=jnp.float32, precision=_HI) + mb3_ref[...]
        match_out[...] = match
        syn = jnp.concatenate([gf, match], axis=1)
        sp = gelu(jnp.dot(syn, sW1_ref[...], preferred_element_type=jnp.float32, precision=_HI) + sb1_ref[...])
        sp = gelu(jnp.dot(sp, sW2_ref[...], preferred_element_type=jnp.float32, precision=_HI) + sb2_ref[...])
        sparams_out[...] = jnp.dot(sp, sW3_ref[...], preferred_element_type=jnp.float32, precision=_HI) + sb3_ref[...]

    pat = pat_ref[...]
    gf = gf_ref[...]
    dots = jax.lax.dot_general(gf, pat, (((1,), (1,)), ((), ())),
                               preferred_element_type=jnp.float32)
    denom = jnp.maximum(gn_ref[...] * pn_ref[...], 1e-8)
    sims = dots / denom
    col = j * BLKM + jax.lax.broadcasted_iota(jnp.int32, (N_B, BLKM), 1)
    valid = col < N_MEM
    sims = jnp.where(valid, sims, _NEG)
    col = jnp.where(valid, col, _BIGI)

    comb_v = jnp.concatenate([vals_scr[...], sims], axis=1)
    comb_i = jnp.concatenate([idxs_scr[...], col], axis=1)
    new_v = jnp.full((N_B, 128), _NEG, jnp.float32)
    new_i = jnp.full((N_B, 128), _BIGI, jnp.int32)
    out_lane = jax.lax.broadcasted_iota(jnp.int32, (N_B, 128), 1)
    for k in range(TOPK):
        m = jnp.max(comb_v, axis=1, keepdims=True)
        is_m = comb_v == m
        sel = jnp.min(jnp.where(is_m, comb_i, _BIGI), axis=1, keepdims=True)
        comb_v = jnp.where(comb_i == sel, _NEG, comb_v)
        new_v = jnp.where(out_lane == k, m, new_v)
        new_i = jnp.where(out_lane == k, sel, new_i)
    vals_scr[...] = new_v
    idxs_scr[...] = new_i

    @pl.when(j == NBLK - 1)
    def _fin():
        vals_out[...] = vals_scr[...]
        idxs_out[...] = idxs_scr[...]


_NC = 2
_NS = 16
_NW = _NC * _NS
_BPW = (N_B * TOPK) // _NW
_QPW = _BPW // TOPK
_LANES = 16


def _sc_gather(idx_hbm, cat_hbm, comp_hbm, catw_hbm, compw_hbm,
               idx_v, cat_v, comp_v, acc_cat, acc_comp, sem):
    wid = lax.axis_index("s") * _NC + lax.axis_index("c")
    base = wid * _BPW
    pltpu.sync_copy(idx_hbm.at[pl.ds(base, _BPW)], idx_v)
    pltpu.async_copy(cat_hbm.at[idx_v], cat_v, sem).wait()
    pltpu.async_copy(comp_hbm.at[idx_v], comp_v, sem).wait()
    inv_k = jnp.float32(1.0 / TOPK)
    for g in range(_QPW):
        for c in range(32 // _LANES):
            acc = cat_v[g * TOPK, pl.ds(c * _LANES, _LANES)]
            for r in range(1, TOPK):
                acc = acc + cat_v[g * TOPK + r, pl.ds(c * _LANES, _LANES)]
            acc_cat[g, pl.ds(c * _LANES, _LANES)] = acc * inv_k
        accc = comp_v[g * TOPK, pl.ds(0, _LANES)]
        for r in range(1, TOPK):
            accc = accc + comp_v[g * TOPK + r, pl.ds(0, _LANES)]
        acc_comp[g, pl.ds(0, _LANES)] = accc * inv_k
    pltpu.sync_copy(acc_cat, catw_hbm.at[pl.ds(wid * _QPW, _QPW)])
    pltpu.sync_copy(acc_comp, compw_hbm.at[pl.ds(wid * _QPW, _QPW)])


_sc_gather_call = functools.partial(
    pl.kernel,
    mesh=plsc.VectorSubcoreMesh(core_axis_name="c", subcore_axis_name="s"),
    compiler_params=pltpu.CompilerParams(use_tc_tiling_on_sc=False),
    out_type=[
        jax.ShapeDtypeStruct((N_B, 32), jnp.float32),
        jax.ShapeDtypeStruct((N_B, 16), jnp.float32),
    ],
    scratch_types=[
        pltpu.VMEM((_BPW,), jnp.int32),
        pltpu.VMEM((_BPW, 32), jnp.float32),
        pltpu.VMEM((_BPW, 16), jnp.float32),
        pltpu.VMEM((_QPW, 32), jnp.float32),
        pltpu.VMEM((_QPW, 16), jnp.float32),
        pltpu.SemaphoreType.DMA,
    ],
)(_sc_gather)


@functools.partial(jax.jit, static_argnames=())
def kernel(features, strategic_patterns, pattern_categories, pattern_complexity,
           mW1, mb1, mW2, mb2, mW3, mb3, sW1, sb1, sW2, sb2, sW3, sb3):
    global_features = jnp.mean(features, axis=(2, 3))
    gn = jnp.linalg.norm(global_features, axis=1, keepdims=True)
    pn = jnp.linalg.norm(strategic_patterns, axis=1, keepdims=True)
    pnT = pn.T

    bias2 = lambda b: b.reshape(1, -1)
    full = lambda shape: pl.BlockSpec(shape, lambda j: tuple(0 for _ in shape))

    vals, idxs, match, sparams = pl.pallas_call(
        _scan_kernel,
        grid=(NBLK,),
        in_specs=[
            full((N_B, D_MODEL)),
            full((N_B, 1)),
            pl.BlockSpec((BLKM, D_MODEL), lambda j: (j, 0)),
            pl.BlockSpec((1, BLKM), lambda j: (0, j)),
            full((D_MODEL, 256)), full((1, 256)),
            full((256, D_MODEL)), full((1, D_MODEL)),
            full((D_MODEL, 128)), full((1, 128)),
            full((256, 256)), full((1, 256)),
            full((256, D_MODEL)), full((1, D_MODEL)),
            full((D_MODEL, 64)), full((1, 64)),
        ],
        out_specs=[
            full((N_B, 128)), full((N_B, 128)),
            full((N_B, 128)), full((N_B, 64)),
        ],
        out_shape=[
            jax.ShapeDtypeStruct((N_B, 128), jnp.float32),
            jax.ShapeDtypeStruct((N_B, 128), jnp.int32),
            jax.ShapeDtypeStruct((N_B, 128), jnp.float32),
            jax.ShapeDtypeStruct((N_B, 64), jnp.float32),
        ],
        scratch_shapes=[
            pltpu.VMEM((N_B, 128), jnp.float32),
            pltpu.VMEM((N_B, 128), jnp.int32),
        ],
    )(global_features, gn, strategic_patterns, pnT,
      mW1, bias2(mb1), mW2, bias2(mb2), mW3, bias2(mb3),
      sW1, bias2(sb1), sW2, bias2(sb2), sW3, bias2(sb3))

    idx_flat = idxs[:, :TOPK].reshape(N_B * TOPK)
    cat_w, comp_w = _sc_gather_call(idx_flat, pattern_categories, pattern_complexity)

    return (vals[:, :TOPK], idxs[:, :TOPK], cat_w, comp_w, sparams, match)
